# baseline (speedup 1.0000x reference)
import functools

import jax
import jax.numpy as jnp
from jax import lax
from jax.experimental import pallas as pl
from jax.experimental.pallas import tpu as pltpu
from jax.experimental.pallas import tpu_sc as plsc


def kernel(x, embedding_table):
    seq_len = x.shape[1]
    table_rows, d_model = embedding_table.shape

    info = plsc.get_sparse_core_info()
    nc, ns = info.num_cores, info.num_subcores
    nw = nc * ns
    rows_per_w = seq_len // nw
    row_elems = seq_len * d_model

    mesh = plsc.VectorSubcoreMesh(core_axis_name="c", subcore_axis_name="s")

    nbuf = 4
    pieces = 16
    chunk = row_elems // pieces
    nchunks = rows_per_w * pieces

    @functools.partial(
        pl.kernel,
        mesh=mesh,
        out_type=jax.ShapeDtypeStruct((seq_len * seq_len * d_model,), jnp.float32),
        scratch_types=[
            pltpu.VMEM((nbuf, chunk), jnp.float32),
            pltpu.SemaphoreType.DMA((nbuf,)),
            pltpu.SemaphoreType.DMA((nbuf,)),
        ],
    )
    def copy_kernel(table_hbm, out_hbm, buf, sem_in, sem_out):
        cid = lax.axis_index("c")
        sid = lax.axis_index("s")
        wid = sid * nc + cid
        row0 = wid * rows_per_w

        def src_of(k):
            r = lax.div(k, pieces)
            p = lax.rem(k, pieces)
            return pl.multiple_of(
                (seq_len - row0 - r) * d_model + p * chunk, 128
            )

        def dst_of(k):
            r = lax.div(k, pieces)
            p = lax.rem(k, pieces)
            return pl.multiple_of((row0 + r) * row_elems + p * chunk, 128)

        def start_in(k, b):
            pltpu.make_async_copy(
                table_hbm.at[pl.ds(src_of(k), chunk)], buf.at[b], sem_in.at[b]
            ).start()

        def wait_in(k, b):
            pltpu.make_async_copy(
                table_hbm.at[pl.ds(src_of(k), chunk)], buf.at[b], sem_in.at[b]
            ).wait()

        def start_out(k, b):
            pltpu.make_async_copy(
                buf.at[b], out_hbm.at[pl.ds(dst_of(k), chunk)], sem_out.at[b]
            ).start()

        def wait_out(k, b):
            pltpu.make_async_copy(
                buf.at[b], out_hbm.at[pl.ds(dst_of(k), chunk)], sem_out.at[b]
            ).wait()

        lag = 2

        def step(k4, carry):
            for b in range(nbuf):
                k = k4 * nbuf + b

                @pl.when(k < nchunks)
                def _in_phase():
                    @pl.when(k >= nbuf)
                    def _drain_prev():
                        wait_out(k - nbuf, b)

                    start_in(k, b)

                j = k - lag
                bj = (b - lag) % nbuf

                @pl.when(jnp.logical_and(j >= 0, j < nchunks))
                def _out_phase():
                    wait_in(j, bj)
                    start_out(j, bj)

            return carry

        niter = (nchunks + lag + nbuf - 1) // nbuf + 1
        lax.fori_loop(0, niter, step, 0, unroll=False)
        for t in range(nbuf):
            j = nchunks - nbuf + t
            wait_out(j, j % nbuf)

    flat = copy_kernel(embedding_table.reshape(-1))
    return flat.reshape(seq_len, seq_len, d_model)


# speedup vs baseline: 1.0003x; 1.0003x over previous
import functools

import jax
import jax.numpy as jnp
from jax import lax
from jax.experimental import pallas as pl
from jax.experimental.pallas import tpu as pltpu
from jax.experimental.pallas import tpu_sc as plsc


def kernel(x, embedding_table):
    seq_len = x.shape[1]
    table_rows, d_model = embedding_table.shape

    info = plsc.get_sparse_core_info()
    nc, ns = info.num_cores, info.num_subcores
    nw = nc * ns
    rows_per_w = seq_len // nw
    row_elems = seq_len * d_model

    mesh = plsc.VectorSubcoreMesh(core_axis_name="c", subcore_axis_name="s")

    nbuf = 2
    pieces = 8
    chunk = row_elems // pieces
    nchunks = rows_per_w * pieces

    @functools.partial(
        pl.kernel,
        mesh=mesh,
        out_type=jax.ShapeDtypeStruct((seq_len * seq_len * d_model,), jnp.float32),
        scratch_types=[
            pltpu.VMEM((nbuf, chunk), jnp.float32),
            pltpu.SemaphoreType.DMA((nbuf,)),
            pltpu.SemaphoreType.DMA((nbuf,)),
        ],
    )
    def copy_kernel(table_hbm, out_hbm, buf, sem_in, sem_out):
        cid = lax.axis_index("c")
        sid = lax.axis_index("s")
        wid = sid * nc + cid
        row0 = wid * rows_per_w

        def src_of(k):
            r = lax.div(k, pieces)
            p = lax.rem(k, pieces)
            return pl.multiple_of(
                (seq_len - row0 - r) * d_model + p * chunk, 128
            )

        def dst_of(k):
            r = lax.div(k, pieces)
            p = lax.rem(k, pieces)
            return pl.multiple_of((row0 + r) * row_elems + p * chunk, 128)

        def start_in(k, b):
            pltpu.make_async_copy(
                table_hbm.at[pl.ds(src_of(k), chunk)], buf.at[b], sem_in.at[b]
            ).start()

        def wait_in(k, b):
            pltpu.make_async_copy(
                table_hbm.at[pl.ds(src_of(k), chunk)], buf.at[b], sem_in.at[b]
            ).wait()

        def start_out(k, b):
            pltpu.make_async_copy(
                buf.at[b], out_hbm.at[pl.ds(dst_of(k), chunk)], sem_out.at[b]
            ).start()

        def wait_out(k, b):
            pltpu.make_async_copy(
                buf.at[b], out_hbm.at[pl.ds(dst_of(k), chunk)], sem_out.at[b]
            ).wait()

        lag = 1

        def step(k4, carry):
            for b in range(nbuf):
                k = k4 * nbuf + b

                @pl.when(k < nchunks)
                def _in_phase():
                    @pl.when(k >= nbuf)
                    def _drain_prev():
                        wait_out(k - nbuf, b)

                    start_in(k, b)

                j = k - lag
                bj = (b - lag) % nbuf

                @pl.when(jnp.logical_and(j >= 0, j < nchunks))
                def _out_phase():
                    wait_in(j, bj)
                    start_out(j, bj)

            return carry

        niter = (nchunks + lag + nbuf - 1) // nbuf + 1
        lax.fori_loop(0, niter, step, 0, unroll=False)
        for t in range(nbuf):
            j = nchunks - nbuf + t
            wait_out(j, j % nbuf)

    flat = copy_kernel(embedding_table.reshape(-1))
    return flat.reshape(seq_len, seq_len, d_model)


# speedup vs baseline: 5.8791x; 5.8776x over previous
import jax
import jax.numpy as jnp
from jax.experimental import pallas as pl
from jax.experimental.pallas import tpu as pltpu


def kernel(x, embedding_table):
    seq_len = x.shape[1]
    table_rows, d_model = embedding_table.shape

    rows_per_block = 2
    num_blocks = seq_len // rows_per_block

    def body(emb_ref, out_ref, scratch_ref):
        b = pl.program_id(0)

        @pl.when(b == 0)
        def _build():
            for cs in range(8):
                scratch_ref[cs] = emb_ref[cs:cs + 2 * seq_len, :]

        for r in range(rows_per_block):
            i = b * rows_per_block + r
            start = seq_len - i
            c = jax.lax.rem(start, 8)
            off = pl.multiple_of(start - c, 8)
            out_ref[r] = scratch_ref[c, pl.ds(off, seq_len), :]

    return pl.pallas_call(
        body,
        grid=(num_blocks,),
        in_specs=[pl.BlockSpec((table_rows, d_model), lambda b: (0, 0))],
        out_specs=pl.BlockSpec(
            (rows_per_block, seq_len, d_model), lambda b: (b, 0, 0)
        ),
        out_shape=jax.ShapeDtypeStruct((seq_len, seq_len, d_model), jnp.float32),
        scratch_shapes=[pltpu.VMEM((8, 2 * seq_len, d_model), jnp.float32)],
    )(embedding_table)


# speedup vs baseline: 5.9362x; 1.0097x over previous
import jax
import jax.numpy as jnp
from jax.experimental import pallas as pl
from jax.experimental.pallas import tpu as pltpu


def kernel(x, embedding_table):
    seq_len = x.shape[1]
    table_rows, d_model = embedding_table.shape

    rows_per_block = 4
    num_blocks = seq_len // rows_per_block

    def body(emb_ref, out_ref, scratch_ref):
        b = pl.program_id(0)

        @pl.when(b == 0)
        def _build():
            for cs in range(8):
                scratch_ref[cs] = emb_ref[cs:cs + 2 * seq_len, :]

        for r in range(rows_per_block):
            i = b * rows_per_block + r
            start = seq_len - i
            c = jax.lax.rem(start, 8)
            off = pl.multiple_of(start - c, 8)
            out_ref[r] = scratch_ref[c, pl.ds(off, seq_len), :]

    return pl.pallas_call(
        body,
        grid=(num_blocks,),
        in_specs=[pl.BlockSpec((table_rows, d_model), lambda b: (0, 0))],
        out_specs=pl.BlockSpec(
            (rows_per_block, seq_len, d_model), lambda b: (b, 0, 0)
        ),
        out_shape=jax.ShapeDtypeStruct((seq_len, seq_len, d_model), jnp.float32),
        scratch_shapes=[pltpu.VMEM((8, 2 * seq_len, d_model), jnp.float32)],
    )(embedding_table)
